# baseline (speedup 1.0000x reference)
import functools
import jax
import jax.numpy as jnp
from jax import lax
from jax.experimental import pallas as pl
from jax.experimental.pallas import tpu as pltpu
from jax.experimental.pallas import tpu_sc as plsc

BATCH = 4
SEQ_LEN = 8192
D_MODEL = 1024
NC, NS, L = 2, 16, 16
NW = NC * NS
ROWS_PER_W = SEQ_LEN // NW
T = 8
N_CHUNKS = ROWS_PER_W // T
N_STEPS = N_CHUNKS * BATCH
VECS_PER_ROW = D_MODEL // L
N_IO = 8
QS = N_STEPS // 8

_mesh = plsc.VectorSubcoreMesh(core_axis_name="c", subcore_axis_name="s")


@functools.partial(
    pl.kernel,
    out_type=jax.ShapeDtypeStruct((BATCH, SEQ_LEN, D_MODEL), jnp.float32),
    mesh=_mesh,
    scratch_types=(
        [pltpu.VMEM((T, D_MODEL), jnp.float32)] * 2
        + [pltpu.VMEM((T, D_MODEL), jnp.float32)] * N_IO
        + [pltpu.SemaphoreType.DMA] * 2
        + [pltpu.SemaphoreType.DMA] * N_IO
        + [pltpu.SemaphoreType.DMA] * N_IO
    ),
)
def _sc_add(in_hbm, tab_hbm, out_hbm, *scratch):
    tabs = scratch[0:2]
    ios = scratch[2:2 + N_IO]
    tab_sems = scratch[2 + N_IO:4 + N_IO]
    in_sems = scratch[4 + N_IO:4 + 2 * N_IO]
    out_sems = scratch[4 + 2 * N_IO:4 + 3 * N_IO]

    wid = lax.axis_index("s") * NC + lax.axis_index("c")
    base = wid * ROWS_PER_W

    def compute(io, tab):
        @plsc.parallel_loop(0, T * VECS_PER_ROW, step=1, unroll=8)
        def _(i):
            r = i // VECS_PER_ROW
            col = (i % VECS_PER_ROW) * L
            plsc.addupdate(io.at[r, pl.ds(col, L)], tab[r, pl.ds(col, L)])

    pltpu.async_copy(tab_hbm.at[pl.ds(base, T)], tabs[0], tab_sems[0])
    for j in range(4):
        pltpu.async_copy(in_hbm.at[j, pl.ds(base, T)], ios[j], in_sems[j])

    def q_body(q, _):
        for j in range(8):
            par = 1 if j >= 4 else 0
            b = j % 4
            ci = 2 * q + par
            row0 = base + ci * T
            j4 = (j + 4) % 8

            if j < 4:
                @pl.when(q > 0)
                def _():
                    pltpu.make_async_copy(
                        ios[j4], out_hbm.at[0, pl.ds(row0, T)],
                        out_sems[j4]).wait()
            else:
                pltpu.make_async_copy(
                    ios[j4], out_hbm.at[0, pl.ds(row0, T)],
                    out_sems[j4]).wait()

            ci_n = ci + 1
            rown = base + ci_n * T
            if j < 4:
                pltpu.async_copy(in_hbm.at[b, pl.ds(rown, T)],
                                 ios[j4], in_sems[j4])
            else:
                @pl.when(q < QS - 1)
                def _():
                    pltpu.async_copy(in_hbm.at[b, pl.ds(rown, T)],
                                     ios[j4], in_sems[j4])

            pltpu.make_async_copy(in_hbm.at[b, pl.ds(row0, T)],
                                  ios[j], in_sems[j]).wait()
            if b == 0:
                pltpu.make_async_copy(tab_hbm.at[pl.ds(row0, T)],
                                      tabs[par], tab_sems[par]).wait()
                if par == 0:
                    pltpu.async_copy(tab_hbm.at[pl.ds(row0 + T, T)],
                                     tabs[1], tab_sems[1])
                else:
                    @pl.when(q < QS - 1)
                    def _():
                        pltpu.async_copy(tab_hbm.at[pl.ds(row0 + T, T)],
                                         tabs[0], tab_sems[0])

            compute(ios[j], tabs[par])

            pltpu.async_copy(ios[j], out_hbm.at[b, pl.ds(row0, T)],
                             out_sems[j])
        return 0

    lax.fori_loop(0, QS, q_body, 0)

    last = base + (N_CHUNKS - 1) * T
    for j in range(4, 8):
        pltpu.make_async_copy(ios[j], out_hbm.at[j - 4, pl.ds(last, T)],
                              out_sems[j]).wait()


def kernel(inputs, pos_table):
    return _sc_add(inputs, pos_table)
